# baseline (speedup 1.0000x reference)
import functools

import jax
import jax.numpy as jnp
from jax.experimental import pallas as pl
from jax.experimental.pallas import tpu as pltpu


def _body(s_ref, t_ref, x_ref, w1_ref, b1_ref, w2_ref, wp_ref, gc_ref,
          o_ref, *, n_types, out_d, hidden):
    x = x_ref[...].astype(jnp.bfloat16)
    h1 = jnp.dot(x, w1_ref[...], preferred_element_type=jnp.float32)
    h1 = jnp.maximum(h1 + b1_ref[...], 0.0).astype(jnp.bfloat16)
    y2 = jnp.dot(h1, w2_ref[...], preferred_element_type=jnp.float32)
    d = y2[:, :hidden]
    y_a = y2[:, hidden:hidden + 1]
    y_b = y2[:, hidden + 1:hidden + 2]
    db = d.astype(jnp.bfloat16)
    y = jnp.dot(db, wp_ref[...],
                preferred_element_type=jnp.float32)

    inv_h = 1.0 / hidden
    sb2 = s_ref[0]
    sb2sq = s_ref[1]
    mu = (y_a + sb2) * inv_h
    s2 = jnp.sum(d * d, axis=-1, keepdims=True) + 2.0 * y_b + sb2sq
    var = s2 * inv_h - mu * mu
    rstd = jax.lax.rsqrt(jnp.maximum(var, 0.0) + 1e-5)

    t = t_ref[...]
    bm = t.shape[0]
    lanes = jax.lax.broadcasted_iota(jnp.int32, (bm, 128), 1)
    onehot = (lanes == t).astype(jnp.bfloat16)
    corr = jnp.dot(onehot, gc_ref[...],
                   preferred_element_type=jnp.float32)
    g_sel = corr[:, :out_d]
    c_sel = corr[:, out_d:2 * out_d]
    bw_sel = corr[:, 2 * out_d:]

    y_sel = y[:, (n_types - 1) * out_d:]
    for tt in range(n_types - 2, -1, -1):
        y_sel = jnp.where(t == tt, y[:, tt * out_d:(tt + 1) * out_d], y_sel)
    o_ref[...] = rstd * (y_sel + bw_sel - mu * g_sel) + c_sel


def kernel(node_latent, node_types, w1, b1, w2, b2, ln_gamma, ln_beta,
           head_w, head_b, *, interpret=False, bm=1000):
    n, latent = node_latent.shape
    hidden = w1.shape[1]
    out_d = head_w.shape[2]
    n_types = head_w.shape[0]
    grid = (n // bm,)

    t2 = node_types.reshape(n, 1)
    b1r = b1.reshape(1, hidden)
    w1b = w1.astype(jnp.bfloat16)

    w2ext = jnp.zeros((hidden, hidden + 128), jnp.float32)
    w2ext = w2ext.at[:, :hidden].set(w2)
    w2ext = w2ext.at[:, hidden].set(w2.sum(axis=1))
    w2ext = w2ext.at[:, hidden + 1].set(w2 @ b2)
    w2b = w2ext.astype(jnp.bfloat16)
    scal = jnp.stack([b2.sum(), (b2 * b2).sum()])

    w_cat = head_w.transpose(1, 0, 2).reshape(hidden, n_types * out_d)
    wp_f32 = ln_gamma[:, None] * w_cat
    wp = wp_f32.astype(jnp.bfloat16)
    g1 = (ln_gamma @ w_cat).reshape(n_types, out_d)
    c_all = (ln_beta @ w_cat).reshape(n_types, out_d) + head_b
    bw = (b2 @ wp_f32).reshape(n_types, out_d)
    gc = jnp.zeros((128, 3 * out_d), jnp.float32)
    gc = gc.at[:n_types, :out_d].set(g1)
    gc = gc.at[:n_types, out_d:2 * out_d].set(c_all)
    gc = gc.at[:n_types, 2 * out_d:].set(bw)
    gcb = gc.astype(jnp.bfloat16)

    return pl.pallas_call(
        functools.partial(_body, n_types=n_types, out_d=out_d, hidden=hidden),
        out_shape=jax.ShapeDtypeStruct((n, out_d), jnp.float32),
        grid=grid,
        in_specs=[
            pl.BlockSpec(memory_space=pltpu.SMEM),
            pl.BlockSpec((bm, 1), lambda i: (i, 0)),
            pl.BlockSpec((bm, latent), lambda i: (i, 0)),
            pl.BlockSpec((latent, hidden), lambda i: (0, 0)),
            pl.BlockSpec((1, hidden), lambda i: (0, 0)),
            pl.BlockSpec((hidden, hidden + 128), lambda i: (0, 0)),
            pl.BlockSpec((hidden, n_types * out_d), lambda i: (0, 0)),
            pl.BlockSpec((128, 3 * out_d), lambda i: (0, 0)),
        ],
        out_specs=pl.BlockSpec((bm, out_d), lambda i: (i, 0)),
        compiler_params=pltpu.CompilerParams(
            dimension_semantics=("parallel",),
            vmem_limit_bytes=56 * 1024 * 1024,
        ),
        name="adaptive_decoder",
        interpret=interpret,
    )(scal, t2, node_latent, w1b, b1r, w2b, wp, gcb)


# speedup vs baseline: 1.1008x; 1.1008x over previous
import functools

import jax
import jax.numpy as jnp
from jax.experimental import pallas as pl
from jax.experimental.pallas import tpu as pltpu


def _body(t_ref, x_ref, w1_ref, b1_ref, w2_ref, b2_ref, wp_ref, gc_ref,
          o_ref, *, n_types, out_d, hidden, n_split):
    full = x_ref.shape[0]
    hm = full // n_split
    for s in range(n_split):
        rows = slice(s * hm, (s + 1) * hm)
        x = x_ref[rows, :].astype(jnp.bfloat16)
        h = jnp.dot(x, w1_ref[...], preferred_element_type=jnp.float32)
        h = jnp.maximum(h + b1_ref[...], 0.0)
        h = jnp.dot(h.astype(jnp.bfloat16), w2_ref[...],
                    preferred_element_type=jnp.float32)
        h = h + b2_ref[...]
        y = jnp.dot(h.astype(jnp.bfloat16), wp_ref[...],
                    preferred_element_type=jnp.float32)
        inv_h = 1.0 / hidden
        mu = jnp.sum(h, axis=-1, keepdims=True) * inv_h
        m2 = jnp.sum(h * h, axis=-1, keepdims=True) * inv_h
        rstd = jax.lax.rsqrt(jnp.maximum(m2 - mu * mu, 0.0) + 1e-5)

        t = t_ref[rows, :]
        lanes = jax.lax.broadcasted_iota(jnp.int32, (hm, 128), 1)
        onehot = (lanes == t).astype(jnp.bfloat16)
        corr = jnp.dot(onehot, gc_ref[...],
                       preferred_element_type=jnp.float32)
        g_sel = corr[:, :out_d]
        c_sel = corr[:, out_d:]

        y_sel = y[:, (n_types - 1) * out_d:]
        for tt in range(n_types - 2, -1, -1):
            y_sel = jnp.where(t == tt, y[:, tt * out_d:(tt + 1) * out_d],
                              y_sel)
        o_ref[rows, :] = rstd * y_sel - (rstd * mu) * g_sel + c_sel


def kernel(node_latent, node_types, w1, b1, w2, b2, ln_gamma, ln_beta,
           head_w, head_b, *, interpret=False, bm=1000, n_split=2):
    n, latent = node_latent.shape
    hidden = w1.shape[1]
    out_d = head_w.shape[2]
    n_types = head_w.shape[0]
    grid = (n // bm,)

    t2 = node_types.reshape(n, 1)
    b1r = b1.reshape(1, hidden)
    b2r = b2.reshape(1, hidden)
    w1b = w1.astype(jnp.bfloat16)
    w2b = w2.astype(jnp.bfloat16)
    w_cat = head_w.transpose(1, 0, 2).reshape(hidden, n_types * out_d)
    wp = (ln_gamma[:, None] * w_cat).astype(jnp.bfloat16)
    g1 = (ln_gamma @ w_cat).reshape(n_types, out_d)
    c_all = (ln_beta @ w_cat).reshape(n_types, out_d) + head_b
    gc = jnp.zeros((128, 2 * out_d), jnp.float32)
    gc = gc.at[:n_types, :out_d].set(g1).at[:n_types, out_d:].set(c_all)
    gcb = gc.astype(jnp.bfloat16)

    return pl.pallas_call(
        functools.partial(_body, n_types=n_types, out_d=out_d, hidden=hidden,
                          n_split=n_split),
        out_shape=jax.ShapeDtypeStruct((n, out_d), jnp.float32),
        grid=grid,
        in_specs=[
            pl.BlockSpec((bm, 1), lambda i: (i, 0)),
            pl.BlockSpec((bm, latent), lambda i: (i, 0)),
            pl.BlockSpec((latent, hidden), lambda i: (0, 0)),
            pl.BlockSpec((1, hidden), lambda i: (0, 0)),
            pl.BlockSpec((hidden, hidden), lambda i: (0, 0)),
            pl.BlockSpec((1, hidden), lambda i: (0, 0)),
            pl.BlockSpec((hidden, n_types * out_d), lambda i: (0, 0)),
            pl.BlockSpec((128, 2 * out_d), lambda i: (0, 0)),
        ],
        out_specs=pl.BlockSpec((bm, out_d), lambda i: (i, 0)),
        compiler_params=pltpu.CompilerParams(
            dimension_semantics=("parallel",),
            vmem_limit_bytes=56 * 1024 * 1024,
        ),
        name="adaptive_decoder",
        interpret=interpret,
    )(t2, node_latent, w1b, b1r, w2b, b2r, wp, gcb)


# speedup vs baseline: 1.1326x; 1.0289x over previous
import functools

import jax
import jax.numpy as jnp
from jax.experimental import pallas as pl
from jax.experimental.pallas import tpu as pltpu


def _body(t_ref, x_ref, w1_ref, b1_ref, w2_ref, b2_ref, wp_ref, gc_ref,
          o_ref, *, n_types, out_d, hidden, n_split):
    full = x_ref.shape[0]
    hm = full // n_split
    for s in range(n_split):
        rows = slice(s * hm, (s + 1) * hm)
        x = x_ref[rows, :].astype(jnp.bfloat16)
        h = jnp.dot(x, w1_ref[...], preferred_element_type=jnp.float32)
        h = jnp.maximum(h + b1_ref[...], 0.0)
        h = jnp.dot(h.astype(jnp.bfloat16), w2_ref[...],
                    preferred_element_type=jnp.float32)
        h = h + b2_ref[...]
        y = jnp.dot(h.astype(jnp.bfloat16), wp_ref[...],
                    preferred_element_type=jnp.float32)
        inv_h = 1.0 / hidden
        mu = jnp.sum(h, axis=-1, keepdims=True) * inv_h
        m2 = jnp.sum(h * h, axis=-1, keepdims=True) * inv_h
        rstd = jax.lax.rsqrt(jnp.maximum(m2 - mu * mu, 0.0) + 1e-5)

        t = t_ref[rows, :]
        lanes = jax.lax.broadcasted_iota(jnp.int32, (hm, 128), 1)
        onehot = (lanes == t).astype(jnp.bfloat16)
        corr = jnp.dot(onehot, gc_ref[...],
                       preferred_element_type=jnp.float32)
        g_sel = corr[:, :out_d]
        c_sel = corr[:, out_d:]

        y_sel = y[:, (n_types - 1) * out_d:]
        for tt in range(n_types - 2, -1, -1):
            y_sel = jnp.where(t == tt, y[:, tt * out_d:(tt + 1) * out_d],
                              y_sel)
        o_ref[rows, :] = rstd * y_sel - (rstd * mu) * g_sel + c_sel


def kernel(node_latent, node_types, w1, b1, w2, b2, ln_gamma, ln_beta,
           head_w, head_b, *, interpret=False, bm=800, n_split=1):
    n, latent = node_latent.shape
    hidden = w1.shape[1]
    out_d = head_w.shape[2]
    n_types = head_w.shape[0]
    grid = (n // bm,)

    t2 = node_types.reshape(n, 1)
    b1r = b1.reshape(1, hidden)
    b2r = b2.reshape(1, hidden)
    w1b = w1.astype(jnp.bfloat16)
    w2b = w2.astype(jnp.bfloat16)
    w_cat = head_w.transpose(1, 0, 2).reshape(hidden, n_types * out_d)
    wp = (ln_gamma[:, None] * w_cat).astype(jnp.bfloat16)
    g1 = (ln_gamma @ w_cat).reshape(n_types, out_d)
    c_all = (ln_beta @ w_cat).reshape(n_types, out_d) + head_b
    gc = jnp.zeros((128, 2 * out_d), jnp.float32)
    gc = gc.at[:n_types, :out_d].set(g1).at[:n_types, out_d:].set(c_all)
    gcb = gc.astype(jnp.bfloat16)

    return pl.pallas_call(
        functools.partial(_body, n_types=n_types, out_d=out_d, hidden=hidden,
                          n_split=n_split),
        out_shape=jax.ShapeDtypeStruct((n, out_d), jnp.float32),
        grid=grid,
        in_specs=[
            pl.BlockSpec((bm, 1), lambda i: (i, 0)),
            pl.BlockSpec((bm, latent), lambda i: (i, 0)),
            pl.BlockSpec((latent, hidden), lambda i: (0, 0)),
            pl.BlockSpec((1, hidden), lambda i: (0, 0)),
            pl.BlockSpec((hidden, hidden), lambda i: (0, 0)),
            pl.BlockSpec((1, hidden), lambda i: (0, 0)),
            pl.BlockSpec((hidden, n_types * out_d), lambda i: (0, 0)),
            pl.BlockSpec((128, 2 * out_d), lambda i: (0, 0)),
        ],
        out_specs=pl.BlockSpec((bm, out_d), lambda i: (i, 0)),
        compiler_params=pltpu.CompilerParams(
            dimension_semantics=("parallel",),
            vmem_limit_bytes=56 * 1024 * 1024,
        ),
        name="adaptive_decoder",
        interpret=interpret,
    )(t2, node_latent, w1b, b1r, w2b, b2r, wp, gcb)
